# baseline (speedup 1.0000x reference)
import functools

import jax
import jax.numpy as jnp
from jax import lax
from jax.experimental import pallas as pl
from jax.experimental.pallas import tpu as pltpu
from jax.experimental.pallas import tpu_sc as plsc

_NC = 2
_NS = 16
_NW = _NC * _NS
_K = 128


def _make_sc_pass(n_rows, n_acc, ch, feat):
    mesh = plsc.VectorSubcoreMesh(core_axis_name="c", subcore_axis_name="s")
    zr = n_acc // _NS
    wr = (n_rows // _NS) // 8 * 8
    wr_last = n_rows - (_NS - 1) * wr

    def body(table, src_hbm, dst_hbm, out, acc, src_v, dst_a, dst_b,
             st_a, st_b, sem_ia, sem_ib, sem_a, sem_b):
        cc = lax.axis_index("c")
        ss = lax.axis_index("s")
        wid = ss * _NC + cc

        def zrow(i, carry):
            for j in range(feat // 16):
                st_a[i, pl.ds(j * 16, 16)] = jnp.zeros((16,), jnp.float32)
            return carry
        lax.fori_loop(0, _K, zrow, 0)
        nfull, rem = zr // _K, zr % _K
        for t in range(nfull):
            pltpu.sync_copy(st_a, acc.at[pl.ds(ss * zr + t * _K, _K), :])
        if rem:
            pltpu.sync_copy(st_a.at[pl.ds(0, rem), :],
                            acc.at[pl.ds(ss * zr + nfull * _K, rem), :])

        pltpu.sync_copy(src_hbm.at[wid], src_v)
        plsc.subcore_barrier()

        def step(i, carry):
            c0 = 2 * i
            c1 = 2 * i + 1
            cp_ia = pltpu.async_copy(
                dst_hbm.at[wid, pl.ds(c0, 1), :], dst_a, sem_ia)
            cp_ib = pltpu.async_copy(
                dst_hbm.at[wid, pl.ds(c1, 1), :], dst_b, sem_ib)
            cp_a = pltpu.async_copy(table.at[src_v.at[c0]], st_a, sem_a)
            cp_b = pltpu.async_copy(table.at[src_v.at[c1]], st_b, sem_b)
            cp_ia.wait()
            cp_a.wait()
            pltpu.sync_copy(st_a, acc.at[dst_a.at[0]], add=True)
            cp_ib.wait()
            cp_b.wait()
            pltpu.sync_copy(st_b, acc.at[dst_b.at[0]], add=True)
            return carry
        lax.fori_loop(0, ch // 2, step, 0)
        plsc.subcore_barrier()

        @pl.when(ss < _NS - 1)
        def _():
            pltpu.sync_copy(acc.at[pl.ds(ss * wr, wr), :],
                            out.at[cc, pl.ds(ss * wr, wr), :])

        @pl.when(ss == _NS - 1)
        def _():
            pltpu.sync_copy(acc.at[pl.ds((_NS - 1) * wr, wr_last), :],
                            out.at[cc, pl.ds((_NS - 1) * wr, wr_last), :])

    scratch = [
        pltpu.VMEM_SHARED((n_acc, feat), jnp.float32),
        pltpu.VMEM((ch, _K), jnp.int32),
        pltpu.VMEM((1, _K), jnp.int32),
        pltpu.VMEM((1, _K), jnp.int32),
        pltpu.VMEM((_K, feat), jnp.float32),
        pltpu.VMEM((_K, feat), jnp.float32),
        pltpu.SemaphoreType.DMA,
        pltpu.SemaphoreType.DMA,
        pltpu.SemaphoreType.DMA,
        pltpu.SemaphoreType.DMA,
    ]
    return pl.kernel(
        body,
        out_type=jax.ShapeDtypeStruct((_NC, n_rows, feat), jnp.float32),
        mesh=mesh,
        scratch_types=scratch,
    )


def _mm_bias_relu(x, w, b, bm):
    m, d = x.shape
    h = w.shape[1]

    def body(x_ref, w_ref, b_ref, o_ref):
        z = jnp.dot(x_ref[:, :], w_ref[:, :],
                    preferred_element_type=jnp.float32) + b_ref[0:1, :]
        o_ref[:, :] = jnp.maximum(z, 0.0)

    return pl.pallas_call(
        body,
        grid=(m // bm,),
        in_specs=[
            pl.BlockSpec((bm, d), lambda i: (i, 0)),
            pl.BlockSpec((d, h), lambda i: (0, 0)),
            pl.BlockSpec((1, h), lambda i: (0, 0)),
        ],
        out_specs=pl.BlockSpec((bm, h), lambda i: (i, 0)),
        out_shape=jax.ShapeDtypeStruct((m, h), jnp.float32),
    )(x, w, b.reshape(1, h))


def _edge_rows(attr, w, b, n_real, bm):
    m, de = attr.shape
    h = w.shape[1]

    def body(a_ref, w_ref, b_ref, o_ref):
        i = pl.program_id(0)
        z = jnp.dot(a_ref[:, :], w_ref[:, :],
                    preferred_element_type=jnp.float32) + b_ref[0:1, :]
        z = jnp.maximum(z, 0.0)
        rows = i * bm + lax.broadcasted_iota(jnp.int32, z.shape, 0)
        o_ref[:, :] = jnp.where(rows < n_real, z, 0.0)

    return pl.pallas_call(
        body,
        grid=(m // bm,),
        in_specs=[
            pl.BlockSpec((bm, de), lambda i: (i, 0)),
            pl.BlockSpec((de, h), lambda i: (0, 0)),
            pl.BlockSpec((1, h), lambda i: (0, 0)),
        ],
        out_specs=pl.BlockSpec((bm, h), lambda i: (i, 0)),
        out_shape=jax.ShapeDtypeStruct((m, h), jnp.float32),
    )(attr, w, b.reshape(1, h))


def _layer_update(hc, pp, ep, w, b, bm):
    m, h = hc.shape

    def body(h_ref, p0_ref, p1_ref, q0_ref, q1_ref, w_ref, b_ref, o_ref):
        s = (p0_ref[:, :] + p1_ref[:, :]) + (q0_ref[:, :] + q1_ref[:, :])
        z = jnp.dot(s, w_ref[:, :],
                    preferred_element_type=jnp.float32) + b_ref[0:1, :]
        o_ref[:, :] = h_ref[:, :] + jnp.maximum(z, 0.0)

    blk = pl.BlockSpec((bm, h), lambda i: (i, 0))
    return pl.pallas_call(
        body,
        grid=(m // bm,),
        in_specs=[blk, blk, blk, blk, blk,
                  pl.BlockSpec((h, h), lambda i: (0, 0)),
                  pl.BlockSpec((1, h), lambda i: (0, 0))],
        out_specs=blk,
        out_shape=jax.ShapeDtypeStruct((m, h), jnp.float32),
    )(hc, pp[0], pp[1], ep[0], ep[1], w, b.reshape(1, h))


def _readout(hc, batch2d, w1, b1, w2, b2, w3, b3, g, beta, ngraphs):
    m, h = hc.shape
    out = w3.shape[1]

    def body(h_ref, bt_ref, w1_ref, b1_ref, w2_ref, b2_ref, w3_ref, b3_ref,
             g_ref, be_ref, o_ref):
        gids = lax.broadcasted_iota(jnp.int32, (ngraphs, m), 0)
        onehot = (bt_ref[:, :] == gids).astype(jnp.float32)
        cnt = jnp.sum(onehot, axis=1, keepdims=True)
        sums = jnp.dot(onehot, h_ref[:, :], preferred_element_type=jnp.float32)
        hg = sums / jnp.maximum(cnt, 1.0)
        z = jnp.maximum(jnp.dot(hg, w1_ref[:, :],
                                preferred_element_type=jnp.float32)
                        + b1_ref[0:1, :], 0.0)
        z = jnp.maximum(jnp.dot(z, w2_ref[:, :],
                                preferred_element_type=jnp.float32)
                        + b2_ref[0:1, :], 0.0)
        z = jnp.dot(z, w3_ref[:, :],
                    preferred_element_type=jnp.float32) + b3_ref[0:1, :]
        mu = jnp.mean(z, axis=-1, keepdims=True)
        var = jnp.mean((z - mu) ** 2, axis=-1, keepdims=True)
        o_ref[:, :] = ((z - mu) / jnp.sqrt(var + 1e-5)) * g_ref[0:1, :] \
            + be_ref[0:1, :]

    return pl.pallas_call(
        body,
        out_shape=jax.ShapeDtypeStruct((ngraphs, out), jnp.float32),
    )(hc, batch2d, w1, b1.reshape(1, -1), w2, b2.reshape(1, -1),
      w3, b3.reshape(1, -1), g.reshape(1, -1), beta.reshape(1, -1))


def kernel(x, edge_index, edge_attr, batch, W_in, b_in, W_e, b_e, W_l, b_l,
           W1, b1, W2, b2, W3, b3, ln_g, ln_b):
    n, d = x.shape
    e = edge_index.shape[1]
    de = edge_attr.shape[1]
    h = W_in.shape[1]
    nlayers = W_l.shape[0]
    ngraphs = 64

    ew_raw = -(-e // _NW)
    ch = -(-ew_raw // _K)
    ch += ch % 2
    ew = ch * _K
    e_pad = _NW * ew
    pad = e_pad - e
    n_acc = -(-n // (8 * _NS)) * (8 * _NS)
    if n_acc == n:
        n_acc += 8 * _NS

    perm = jnp.argsort(edge_index[0])
    src_s = edge_index[0][perm]
    dst_s = edge_index[1][perm]
    src_p = jnp.concatenate(
        [src_s, jnp.zeros((pad,), jnp.int32)]).reshape(_NW, ch, _K)
    dst_p = jnp.concatenate(
        [dst_s, jnp.full((pad,), n, jnp.int32)]).reshape(_NW, ch, _K)
    iota_p = jnp.arange(e_pad, dtype=jnp.int32).reshape(_NW, ch, _K)
    attr_p = jnp.concatenate(
        [edge_attr, jnp.zeros((pad, de), edge_attr.dtype)])

    h0 = _mm_bias_relu(x, W_in, b_in, bm=1000)
    erows = _edge_rows(attr_p, W_e, b_e, e, bm=2048)

    sc_e = _make_sc_pass(n, n_acc, ch, h)
    ep = sc_e(erows, iota_p, dst_p)

    sc_h = _make_sc_pass(n, n_acc, ch, h)
    hc = h0
    for l in range(nlayers):
        pp = sc_h(hc, src_p, dst_p)
        hc = _layer_update(hc, pp, ep, W_l[l], b_l[l], bm=1000)

    return _readout(hc, batch.reshape(1, n), W1, b1, W2, b2, W3, b3,
                    ln_g, ln_b, ngraphs)


# speedup vs baseline: 1.2616x; 1.2616x over previous
import functools

import jax
import jax.numpy as jnp
from jax import lax
from jax.experimental import pallas as pl
from jax.experimental.pallas import tpu as pltpu
from jax.experimental.pallas import tpu_sc as plsc

_NC = 2
_NS = 16
_NW = _NC * _NS
_K = 64
_NBUF = 4


def _make_sc_pass(n_rows, n_acc, ch, feat):
    mesh = plsc.VectorSubcoreMesh(core_axis_name="c", subcore_axis_name="s")
    zr = n_acc // _NS
    wr = (n_rows // _NS) // 8 * 8
    wr_last = n_rows - (_NS - 1) * wr

    def body(table, sd_hbm, out, acc, *rest):
        sd = rest[0:_NBUF]
        st = rest[_NBUF:2 * _NBUF]
        isem = rest[2 * _NBUF:3 * _NBUF]
        gsem = rest[3 * _NBUF:4 * _NBUF]
        cc = lax.axis_index("c")
        ss = lax.axis_index("s")
        wid = ss * _NC + cc

        st0 = st[0]

        def zrow(i, carry):
            for j in range(feat // 16):
                st0[i, pl.ds(j * 16, 16)] = jnp.zeros((16,), jnp.float32)
            return carry
        lax.fori_loop(0, _K, zrow, 0)
        nfull, rem = zr // _K, zr % _K
        for t in range(nfull):
            pltpu.sync_copy(st0, acc.at[pl.ds(ss * zr + t * _K, _K), :])
        if rem:
            pltpu.sync_copy(st0.at[pl.ds(0, rem), :],
                            acc.at[pl.ds(ss * zr + nfull * _K, rem), :])
        plsc.subcore_barrier()

        def issue_idx(c, slot):
            pltpu.async_copy(sd_hbm.at[wid, c], sd[slot], isem[slot])

        def wait_idx(slot):
            pltpu.make_async_copy(sd_hbm.at[wid, 0], sd[slot],
                                  isem[slot]).wait()

        def issue_gather(slot):
            pltpu.async_copy(table.at[sd[slot].at[0]], st[slot], gsem[slot])

        def wait_gather(slot):
            pltpu.make_async_copy(table.at[sd[slot].at[0]], st[slot],
                                  gsem[slot]).wait()

        for b in range(_NBUF):
            issue_idx(b, b)
        for b in range(2):
            wait_idx(b)
            issue_gather(b)

        def group(i, carry):
            for b in range(_NBUF):
                c = _NBUF * i + b
                c2 = c + 2

                @pl.when(c2 < ch)
                def _():
                    wait_idx((b + 2) % _NBUF)
                    issue_gather((b + 2) % _NBUF)
                wait_gather(b)
                pltpu.sync_copy(st[b], acc.at[sd[b].at[1]], add=True)

                @pl.when(c + _NBUF < ch)
                def _():
                    issue_idx(c + _NBUF, b)
            return carry
        lax.fori_loop(0, ch // _NBUF, group, 0)
        plsc.subcore_barrier()

        @pl.when(ss < _NS - 1)
        def _():
            pltpu.sync_copy(acc.at[pl.ds(ss * wr, wr), :],
                            out.at[cc, pl.ds(ss * wr, wr), :])

        @pl.when(ss == _NS - 1)
        def _():
            pltpu.sync_copy(acc.at[pl.ds((_NS - 1) * wr, wr_last), :],
                            out.at[cc, pl.ds((_NS - 1) * wr, wr_last), :])

    scratch = (
        [pltpu.VMEM_SHARED((n_acc, feat), jnp.float32)]
        + [pltpu.VMEM((2, _K), jnp.int32)] * _NBUF
        + [pltpu.VMEM((_K, feat), jnp.float32)] * _NBUF
        + [pltpu.SemaphoreType.DMA] * (2 * _NBUF)
    )
    return pl.kernel(
        body,
        out_type=jax.ShapeDtypeStruct((_NC, n_rows, feat), jnp.float32),
        mesh=mesh,
        scratch_types=scratch,
    )


def _mm_bias_relu(x, w, b, bm):
    m, d = x.shape
    h = w.shape[1]

    def body(x_ref, w_ref, b_ref, o_ref):
        z = jnp.dot(x_ref[:, :], w_ref[:, :],
                    preferred_element_type=jnp.float32) + b_ref[0:1, :]
        o_ref[:, :] = jnp.maximum(z, 0.0)

    return pl.pallas_call(
        body,
        grid=(m // bm,),
        in_specs=[
            pl.BlockSpec((bm, d), lambda i: (i, 0)),
            pl.BlockSpec((d, h), lambda i: (0, 0)),
            pl.BlockSpec((1, h), lambda i: (0, 0)),
        ],
        out_specs=pl.BlockSpec((bm, h), lambda i: (i, 0)),
        out_shape=jax.ShapeDtypeStruct((m, h), jnp.float32),
    )(x, w, b.reshape(1, h))


def _edge_rows(attr, w, b, n_real, bm):
    m, de = attr.shape
    h = w.shape[1]

    def body(a_ref, w_ref, b_ref, o_ref):
        i = pl.program_id(0)
        z = jnp.dot(a_ref[:, :], w_ref[:, :],
                    preferred_element_type=jnp.float32) + b_ref[0:1, :]
        z = jnp.maximum(z, 0.0)
        rows = i * bm + lax.broadcasted_iota(jnp.int32, z.shape, 0)
        o_ref[:, :] = jnp.where(rows < n_real, z, 0.0)

    return pl.pallas_call(
        body,
        grid=(m // bm,),
        in_specs=[
            pl.BlockSpec((bm, de), lambda i: (i, 0)),
            pl.BlockSpec((de, h), lambda i: (0, 0)),
            pl.BlockSpec((1, h), lambda i: (0, 0)),
        ],
        out_specs=pl.BlockSpec((bm, h), lambda i: (i, 0)),
        out_shape=jax.ShapeDtypeStruct((m, h), jnp.float32),
    )(attr, w, b.reshape(1, h))


def _layer_update(hc, pp, ep, w, b, bm):
    m, h = hc.shape

    def body(h_ref, p0_ref, p1_ref, q0_ref, q1_ref, w_ref, b_ref, o_ref):
        s = (p0_ref[:, :] + p1_ref[:, :]) + (q0_ref[:, :] + q1_ref[:, :])
        z = jnp.dot(s, w_ref[:, :],
                    preferred_element_type=jnp.float32) + b_ref[0:1, :]
        o_ref[:, :] = h_ref[:, :] + jnp.maximum(z, 0.0)

    blk = pl.BlockSpec((bm, h), lambda i: (i, 0))
    return pl.pallas_call(
        body,
        grid=(m // bm,),
        in_specs=[blk, blk, blk, blk, blk,
                  pl.BlockSpec((h, h), lambda i: (0, 0)),
                  pl.BlockSpec((1, h), lambda i: (0, 0))],
        out_specs=blk,
        out_shape=jax.ShapeDtypeStruct((m, h), jnp.float32),
    )(hc, pp[0], pp[1], ep[0], ep[1], w, b.reshape(1, h))


def _readout(hc, batch2d, w1, b1, w2, b2, w3, b3, g, beta, ngraphs):
    m, h = hc.shape
    out = w3.shape[1]

    def body(h_ref, bt_ref, w1_ref, b1_ref, w2_ref, b2_ref, w3_ref, b3_ref,
             g_ref, be_ref, o_ref):
        gids = lax.broadcasted_iota(jnp.int32, (ngraphs, m), 0)
        onehot = (bt_ref[:, :] == gids).astype(jnp.float32)
        cnt = jnp.sum(onehot, axis=1, keepdims=True)
        sums = jnp.dot(onehot, h_ref[:, :], preferred_element_type=jnp.float32)
        hg = sums / jnp.maximum(cnt, 1.0)
        z = jnp.maximum(jnp.dot(hg, w1_ref[:, :],
                                preferred_element_type=jnp.float32)
                        + b1_ref[0:1, :], 0.0)
        z = jnp.maximum(jnp.dot(z, w2_ref[:, :],
                                preferred_element_type=jnp.float32)
                        + b2_ref[0:1, :], 0.0)
        z = jnp.dot(z, w3_ref[:, :],
                    preferred_element_type=jnp.float32) + b3_ref[0:1, :]
        mu = jnp.mean(z, axis=-1, keepdims=True)
        var = jnp.mean((z - mu) ** 2, axis=-1, keepdims=True)
        o_ref[:, :] = ((z - mu) / jnp.sqrt(var + 1e-5)) * g_ref[0:1, :] \
            + be_ref[0:1, :]

    return pl.pallas_call(
        body,
        out_shape=jax.ShapeDtypeStruct((ngraphs, out), jnp.float32),
    )(hc, batch2d, w1, b1.reshape(1, -1), w2, b2.reshape(1, -1),
      w3, b3.reshape(1, -1), g.reshape(1, -1), beta.reshape(1, -1))


def kernel(x, edge_index, edge_attr, batch, W_in, b_in, W_e, b_e, W_l, b_l,
           W1, b1, W2, b2, W3, b3, ln_g, ln_b):
    n, d = x.shape
    e = edge_index.shape[1]
    de = edge_attr.shape[1]
    h = W_in.shape[1]
    nlayers = W_l.shape[0]
    ngraphs = 64

    ew_raw = -(-e // _NW)
    ch = -(-ew_raw // _K)
    ch = -(-ch // _NBUF) * _NBUF
    ew = ch * _K
    e_pad = _NW * ew
    pad = e_pad - e
    n_acc = -(-n // (8 * _NS)) * (8 * _NS)
    if n_acc == n:
        n_acc += 8 * _NS

    src_p = jnp.concatenate(
        [edge_index[0], jnp.zeros((pad,), jnp.int32)]).reshape(_NW, ch, 1, _K)
    dst_p = jnp.concatenate(
        [edge_index[1], jnp.full((pad,), n, jnp.int32)]).reshape(_NW, ch, 1, _K)
    sd_p = jnp.concatenate([src_p, dst_p], axis=2)
    iota_p = jnp.arange(e_pad, dtype=jnp.int32).reshape(_NW, ch, 1, _K)
    sd_e = jnp.concatenate([iota_p, dst_p], axis=2)
    attr_p = jnp.concatenate(
        [edge_attr, jnp.zeros((pad, de), edge_attr.dtype)])

    h0 = _mm_bias_relu(x, W_in, b_in, bm=1000)
    erows = _edge_rows(attr_p, W_e, b_e, e, bm=2048)

    sc_e = _make_sc_pass(n, n_acc, ch, h)
    ep = sc_e(erows, sd_e)

    sc_h = _make_sc_pass(n, n_acc, ch, h)
    hc = h0
    for l in range(nlayers):
        pp = sc_h(hc, sd_p)
        hc = _layer_update(hc, pp, ep, W_l[l], b_l[l], bm=1000)

    return _readout(hc, batch.reshape(1, n), W1, b1, W2, b2, W3, b3,
                    ln_g, ln_b, ngraphs)


# speedup vs baseline: 1.3368x; 1.0597x over previous
import functools

import jax
import jax.numpy as jnp
from jax import lax
from jax.experimental import pallas as pl
from jax.experimental.pallas import tpu as pltpu
from jax.experimental.pallas import tpu_sc as plsc

_NC = 2
_NS = 16
_NW = _NC * _NS
_K = 64
_NBUF = 4


def _make_sc_pass(n_rows, n_acc, g0, g1, feat):
    mesh = plsc.VectorSubcoreMesh(core_axis_name="c", subcore_axis_name="s")
    zr = n_acc // _NS
    wr = (n_rows // _NS) // 8 * 8
    wr_last = n_rows - (_NS - 1) * wr

    def body(table, sd_hbm, out, acc, *rest):
        sd = rest[0:_NBUF]
        st = rest[_NBUF:2 * _NBUF]
        isem = rest[2 * _NBUF:3 * _NBUF]
        gsem = rest[3 * _NBUF:4 * _NBUF]
        cc = lax.axis_index("c")
        ss = lax.axis_index("s")

        st0 = st[0]

        def zrow(i, carry):
            for j in range(feat // 16):
                st0[i, pl.ds(j * 16, 16)] = jnp.zeros((16,), jnp.float32)
            return carry
        lax.fori_loop(0, _K, zrow, 0)
        nfull, rem = zr // _K, zr % _K
        for t in range(nfull):
            pltpu.sync_copy(st0, acc.at[pl.ds(ss * zr + t * _K, _K), :])
        if rem:
            pltpu.sync_copy(st0.at[pl.ds(0, rem), :],
                            acc.at[pl.ds(ss * zr + nfull * _K, rem), :])
        plsc.subcore_barrier()

        def issue_idx(c, slot):
            pltpu.async_copy(sd_hbm.at[c], sd[slot], isem[slot])

        def wait_idx(slot):
            pltpu.make_async_copy(sd_hbm.at[0], sd[slot],
                                  isem[slot]).wait()

        def issue_gather(slot):
            pltpu.async_copy(table.at[sd[slot].at[0]], st[slot], gsem[slot])

        def wait_gather(slot):
            pltpu.make_async_copy(table.at[sd[slot].at[0]], st[slot],
                                  gsem[slot]).wait()

        def run(base, g):
            for b in range(_NBUF):
                issue_idx(base + b, b)
            for b in range(2):
                wait_idx(b)
                issue_gather(b)

            def group(i, carry):
                for b in range(_NBUF):
                    c = _NBUF * i + b
                    c2 = c + 2

                    @pl.when(c2 < g)
                    def _():
                        wait_idx((b + 2) % _NBUF)
                        issue_gather((b + 2) % _NBUF)
                    wait_gather(b)
                    pltpu.sync_copy(st[b], acc.at[sd[b].at[1]], add=True)

                    @pl.when(c + _NBUF < g)
                    def _():
                        issue_idx(base + c + _NBUF, b)
                return carry
            lax.fori_loop(0, g // _NBUF, group, 0)

        @pl.when(cc == 0)
        def _():
            run(ss * g0, g0)

        @pl.when(cc == 1)
        def _():
            run(_NS * g0 + ss * g1, g1)
        plsc.subcore_barrier()

        @pl.when(ss < _NS - 1)
        def _():
            pltpu.sync_copy(acc.at[pl.ds(ss * wr, wr), :],
                            out.at[cc, pl.ds(ss * wr, wr), :])

        @pl.when(ss == _NS - 1)
        def _():
            pltpu.sync_copy(acc.at[pl.ds((_NS - 1) * wr, wr_last), :],
                            out.at[cc, pl.ds((_NS - 1) * wr, wr_last), :])

    scratch = (
        [pltpu.VMEM_SHARED((n_acc, feat), jnp.float32)]
        + [pltpu.VMEM((2, _K), jnp.int32)] * _NBUF
        + [pltpu.VMEM((_K, feat), jnp.float32)] * _NBUF
        + [pltpu.SemaphoreType.DMA] * (2 * _NBUF)
    )
    return pl.kernel(
        body,
        out_type=jax.ShapeDtypeStruct((_NC, n_rows, feat), jnp.float32),
        mesh=mesh,
        scratch_types=scratch,
    )


def _mm_bias_relu(x, w, b, bm):
    m, d = x.shape
    h = w.shape[1]

    def body(x_ref, w_ref, b_ref, o_ref):
        z = jnp.dot(x_ref[:, :], w_ref[:, :],
                    preferred_element_type=jnp.float32) + b_ref[0:1, :]
        o_ref[:, :] = jnp.maximum(z, 0.0)

    return pl.pallas_call(
        body,
        grid=(m // bm,),
        in_specs=[
            pl.BlockSpec((bm, d), lambda i: (i, 0)),
            pl.BlockSpec((d, h), lambda i: (0, 0)),
            pl.BlockSpec((1, h), lambda i: (0, 0)),
        ],
        out_specs=pl.BlockSpec((bm, h), lambda i: (i, 0)),
        out_shape=jax.ShapeDtypeStruct((m, h), jnp.float32),
    )(x, w, b.reshape(1, h))


def _edge_rows(attr, w, b, n_real, bm):
    m, de = attr.shape
    h = w.shape[1]

    def body(a_ref, w_ref, b_ref, o_ref):
        i = pl.program_id(0)
        z = jnp.dot(a_ref[:, :], w_ref[:, :],
                    preferred_element_type=jnp.float32) + b_ref[0:1, :]
        z = jnp.maximum(z, 0.0)
        rows = i * bm + lax.broadcasted_iota(jnp.int32, z.shape, 0)
        o_ref[:, :] = jnp.where(rows < n_real, z, 0.0)

    return pl.pallas_call(
        body,
        grid=(m // bm,),
        in_specs=[
            pl.BlockSpec((bm, de), lambda i: (i, 0)),
            pl.BlockSpec((de, h), lambda i: (0, 0)),
            pl.BlockSpec((1, h), lambda i: (0, 0)),
        ],
        out_specs=pl.BlockSpec((bm, h), lambda i: (i, 0)),
        out_shape=jax.ShapeDtypeStruct((m, h), jnp.float32),
    )(attr, w, b.reshape(1, h))


def _layer_update(hc, pp, ep, w, b, bm):
    m, h = hc.shape

    def body(h_ref, p0_ref, p1_ref, q0_ref, q1_ref, w_ref, b_ref, o_ref):
        s = (p0_ref[:, :] + p1_ref[:, :]) + (q0_ref[:, :] + q1_ref[:, :])
        z = jnp.dot(s, w_ref[:, :],
                    preferred_element_type=jnp.float32) + b_ref[0:1, :]
        o_ref[:, :] = h_ref[:, :] + jnp.maximum(z, 0.0)

    blk = pl.BlockSpec((bm, h), lambda i: (i, 0))
    return pl.pallas_call(
        body,
        grid=(m // bm,),
        in_specs=[blk, blk, blk, blk, blk,
                  pl.BlockSpec((h, h), lambda i: (0, 0)),
                  pl.BlockSpec((1, h), lambda i: (0, 0))],
        out_specs=blk,
        out_shape=jax.ShapeDtypeStruct((m, h), jnp.float32),
    )(hc, pp[0], pp[1], ep[0], ep[1], w, b.reshape(1, h))


def _readout(hc, batch2d, w1, b1, w2, b2, w3, b3, g, beta, ngraphs):
    m, h = hc.shape
    out = w3.shape[1]

    def body(h_ref, bt_ref, w1_ref, b1_ref, w2_ref, b2_ref, w3_ref, b3_ref,
             g_ref, be_ref, o_ref):
        gids = lax.broadcasted_iota(jnp.int32, (ngraphs, m), 0)
        onehot = (bt_ref[:, :] == gids).astype(jnp.float32)
        cnt = jnp.sum(onehot, axis=1, keepdims=True)
        sums = jnp.dot(onehot, h_ref[:, :], preferred_element_type=jnp.float32)
        hg = sums / jnp.maximum(cnt, 1.0)
        z = jnp.maximum(jnp.dot(hg, w1_ref[:, :],
                                preferred_element_type=jnp.float32)
                        + b1_ref[0:1, :], 0.0)
        z = jnp.maximum(jnp.dot(z, w2_ref[:, :],
                                preferred_element_type=jnp.float32)
                        + b2_ref[0:1, :], 0.0)
        z = jnp.dot(z, w3_ref[:, :],
                    preferred_element_type=jnp.float32) + b3_ref[0:1, :]
        mu = jnp.mean(z, axis=-1, keepdims=True)
        var = jnp.mean((z - mu) ** 2, axis=-1, keepdims=True)
        o_ref[:, :] = ((z - mu) / jnp.sqrt(var + 1e-5)) * g_ref[0:1, :] \
            + be_ref[0:1, :]

    return pl.pallas_call(
        body,
        out_shape=jax.ShapeDtypeStruct((ngraphs, out), jnp.float32),
    )(hc, batch2d, w1, b1.reshape(1, -1), w2, b2.reshape(1, -1),
      w3, b3.reshape(1, -1), g.reshape(1, -1), beta.reshape(1, -1))


def kernel(x, edge_index, edge_attr, batch, W_in, b_in, W_e, b_e, W_l, b_l,
           W1, b1, W2, b2, W3, b3, ln_g, ln_b):
    n, d = x.shape
    e = edge_index.shape[1]
    de = edge_attr.shape[1]
    h = W_in.shape[1]
    nlayers = W_l.shape[0]
    ngraphs = 64

    s_tot = -(-e // (_NS * _K))
    s_tot = -(-s_tot // (2 * _NBUF)) * (2 * _NBUF)
    cht = _NS * s_tot
    e_pad = cht * _K
    pad = e_pad - e
    n_acc = -(-n // (8 * _NS)) * (8 * _NS)
    if n_acc == n:
        n_acc += 8 * _NS

    src_p = jnp.concatenate(
        [edge_index[0], jnp.zeros((pad,), jnp.int32)]).reshape(cht, 1, _K)
    dst_p = jnp.concatenate(
        [edge_index[1], jnp.full((pad,), n, jnp.int32)]).reshape(cht, 1, _K)
    sd_p = jnp.concatenate([src_p, dst_p], axis=1)
    iota_p = jnp.arange(e_pad, dtype=jnp.int32).reshape(cht, 1, _K)
    sd_e = jnp.concatenate([iota_p, dst_p], axis=1)
    attr_p = jnp.concatenate(
        [edge_attr, jnp.zeros((pad, de), edge_attr.dtype)])

    h0 = _mm_bias_relu(x, W_in, b_in, bm=1000)
    erows = _edge_rows(attr_p, W_e, b_e, e, bm=2048)

    sc_e = _make_sc_pass(n, n_acc, s_tot // 2, s_tot - s_tot // 2, h)
    ep = sc_e(erows, sd_e)

    g0 = (2 * s_tot // 3) // _NBUF * _NBUF
    sc_h = _make_sc_pass(n, n_acc, g0, s_tot - g0, h)
    hc = h0
    for l in range(nlayers):
        pp = sc_h(hc, sd_p)
        hc = _layer_update(hc, pp, ep, W_l[l], b_l[l], bm=1000)

    return _readout(hc, batch.reshape(1, n), W1, b1, W2, b2, W3, b3,
                    ln_g, ln_b, ngraphs)


# speedup vs baseline: 1.3632x; 1.0197x over previous
import functools

import jax
import jax.numpy as jnp
from jax import lax
from jax.experimental import pallas as pl
from jax.experimental.pallas import tpu as pltpu
from jax.experimental.pallas import tpu_sc as plsc

_NC = 2
_NS = 16
_NW = _NC * _NS
_K = 64
_NBUF = 4


def _make_sc_pass(n_rows, n_acc, g0, g1, feat):
    mesh = plsc.VectorSubcoreMesh(core_axis_name="c", subcore_axis_name="s")
    zr = n_acc // _NS
    wr = (n_rows // _NS) // 8 * 8
    wr_last = n_rows - (_NS - 1) * wr

    def body(table, sd_hbm, out, acc, *rest):
        sd = rest[0:_NBUF]
        st = rest[_NBUF:2 * _NBUF]
        isem = rest[2 * _NBUF:3 * _NBUF]
        gsem = rest[3 * _NBUF:4 * _NBUF]
        cc = lax.axis_index("c")
        ss = lax.axis_index("s")

        st0 = st[0]

        def zrow(i, carry):
            for j in range(feat // 16):
                st0[i, pl.ds(j * 16, 16)] = jnp.zeros((16,), jnp.float32)
            return carry
        lax.fori_loop(0, _K, zrow, 0)
        nfull, rem = zr // _K, zr % _K
        for t in range(nfull):
            pltpu.sync_copy(st0, acc.at[pl.ds(ss * zr + t * _K, _K), :])
        if rem:
            pltpu.sync_copy(st0.at[pl.ds(0, rem), :],
                            acc.at[pl.ds(ss * zr + nfull * _K, rem), :])
        plsc.subcore_barrier()

        def issue_idx(c, slot):
            pltpu.async_copy(sd_hbm.at[c], sd[slot], isem[slot])

        def wait_idx(slot):
            pltpu.make_async_copy(sd_hbm.at[0], sd[slot],
                                  isem[slot]).wait()

        def issue_gather(slot):
            pltpu.async_copy(table.at[sd[slot].at[0]], st[slot], gsem[slot])

        def wait_gather(slot):
            pltpu.make_async_copy(table.at[sd[slot].at[0]], st[slot],
                                  gsem[slot]).wait()

        def run(base, g):
            for b in range(_NBUF):
                issue_idx(base + b, b)
            for b in range(2):
                wait_idx(b)
                issue_gather(b)

            def group(i, carry):
                for b in range(_NBUF):
                    c = _NBUF * i + b
                    c2 = c + 2

                    @pl.when(c2 < g)
                    def _():
                        wait_idx((b + 2) % _NBUF)
                        issue_gather((b + 2) % _NBUF)
                    wait_gather(b)
                    pltpu.sync_copy(st[b], acc.at[sd[b].at[1]], add=True)

                    @pl.when(c + _NBUF < g)
                    def _():
                        issue_idx(base + c + _NBUF, b)
                return carry
            lax.fori_loop(0, g // _NBUF, group, 0)

        @pl.when(cc == 0)
        def _():
            run(ss * g0, g0)

        @pl.when(cc == 1)
        def _():
            run(_NS * g0 + ss * g1, g1)
        plsc.subcore_barrier()

        @pl.when(ss < _NS - 1)
        def _():
            pltpu.sync_copy(acc.at[pl.ds(ss * wr, wr), :],
                            out.at[cc, pl.ds(ss * wr, wr), :])

        @pl.when(ss == _NS - 1)
        def _():
            pltpu.sync_copy(acc.at[pl.ds((_NS - 1) * wr, wr_last), :],
                            out.at[cc, pl.ds((_NS - 1) * wr, wr_last), :])

    scratch = (
        [pltpu.VMEM_SHARED((n_acc, feat), jnp.float32)]
        + [pltpu.VMEM((2, _K), jnp.int32)] * _NBUF
        + [pltpu.VMEM((_K, feat), jnp.float32)] * _NBUF
        + [pltpu.SemaphoreType.DMA] * (2 * _NBUF)
    )
    return pl.kernel(
        body,
        out_type=jax.ShapeDtypeStruct((_NC, n_rows, feat), jnp.float32),
        mesh=mesh,
        scratch_types=scratch,
    )


def _mm_bias_relu(x, w, b, bm):
    m, d = x.shape
    h = w.shape[1]

    def body(x_ref, w_ref, b_ref, o_ref):
        z = jnp.dot(x_ref[:, :], w_ref[:, :],
                    preferred_element_type=jnp.float32) + b_ref[0:1, :]
        o_ref[:, :] = jnp.maximum(z, 0.0)

    return pl.pallas_call(
        body,
        grid=(m // bm,),
        in_specs=[
            pl.BlockSpec((bm, d), lambda i: (i, 0)),
            pl.BlockSpec((d, h), lambda i: (0, 0)),
            pl.BlockSpec((1, h), lambda i: (0, 0)),
        ],
        out_specs=pl.BlockSpec((bm, h), lambda i: (i, 0)),
        out_shape=jax.ShapeDtypeStruct((m, h), jnp.float32),
    )(x, w, b.reshape(1, h))


def _edge_rows(attr, w, b, n_real, bm):
    m, de = attr.shape
    h = w.shape[1]

    def body(a_ref, w_ref, b_ref, o_ref):
        i = pl.program_id(0)
        z = jnp.dot(a_ref[:, :], w_ref[:, :],
                    preferred_element_type=jnp.float32) + b_ref[0:1, :]
        z = jnp.maximum(z, 0.0)
        rows = i * bm + lax.broadcasted_iota(jnp.int32, z.shape, 0)
        o_ref[:, :] = jnp.where(rows < n_real, z, 0.0)

    return pl.pallas_call(
        body,
        grid=(m // bm,),
        in_specs=[
            pl.BlockSpec((bm, de), lambda i: (i, 0)),
            pl.BlockSpec((de, h), lambda i: (0, 0)),
            pl.BlockSpec((1, h), lambda i: (0, 0)),
        ],
        out_specs=pl.BlockSpec((bm, h), lambda i: (i, 0)),
        out_shape=jax.ShapeDtypeStruct((m, h), jnp.float32),
    )(attr, w, b.reshape(1, h))


def _layer_update(hc, pp, ep, w, b, bm):
    m, h = hc.shape

    def body(h_ref, p0_ref, p1_ref, q0_ref, q1_ref, w_ref, b_ref, o_ref):
        s = (p0_ref[:, :] + p1_ref[:, :]) + (q0_ref[:, :] + q1_ref[:, :])
        z = jnp.dot(s, w_ref[:, :],
                    preferred_element_type=jnp.float32) + b_ref[0:1, :]
        o_ref[:, :] = h_ref[:, :] + jnp.maximum(z, 0.0)

    blk = pl.BlockSpec((bm, h), lambda i: (i, 0))
    return pl.pallas_call(
        body,
        grid=(m // bm,),
        in_specs=[blk, blk, blk, blk, blk,
                  pl.BlockSpec((h, h), lambda i: (0, 0)),
                  pl.BlockSpec((1, h), lambda i: (0, 0))],
        out_specs=blk,
        out_shape=jax.ShapeDtypeStruct((m, h), jnp.float32),
    )(hc, pp[0], pp[1], ep[0], ep[1], w, b.reshape(1, h))


def _readout(hc, batch2d, w1, b1, w2, b2, w3, b3, g, beta, ngraphs):
    m, h = hc.shape
    out = w3.shape[1]

    def body(h_ref, bt_ref, w1_ref, b1_ref, w2_ref, b2_ref, w3_ref, b3_ref,
             g_ref, be_ref, o_ref):
        gids = lax.broadcasted_iota(jnp.int32, (ngraphs, m), 0)
        onehot = (bt_ref[:, :] == gids).astype(jnp.float32)
        cnt = jnp.sum(onehot, axis=1, keepdims=True)
        sums = jnp.dot(onehot, h_ref[:, :], preferred_element_type=jnp.float32)
        hg = sums / jnp.maximum(cnt, 1.0)
        z = jnp.maximum(jnp.dot(hg, w1_ref[:, :],
                                preferred_element_type=jnp.float32)
                        + b1_ref[0:1, :], 0.0)
        z = jnp.maximum(jnp.dot(z, w2_ref[:, :],
                                preferred_element_type=jnp.float32)
                        + b2_ref[0:1, :], 0.0)
        z = jnp.dot(z, w3_ref[:, :],
                    preferred_element_type=jnp.float32) + b3_ref[0:1, :]
        mu = jnp.mean(z, axis=-1, keepdims=True)
        var = jnp.mean((z - mu) ** 2, axis=-1, keepdims=True)
        o_ref[:, :] = ((z - mu) / jnp.sqrt(var + 1e-5)) * g_ref[0:1, :] \
            + be_ref[0:1, :]

    return pl.pallas_call(
        body,
        out_shape=jax.ShapeDtypeStruct((ngraphs, out), jnp.float32),
    )(hc, batch2d, w1, b1.reshape(1, -1), w2, b2.reshape(1, -1),
      w3, b3.reshape(1, -1), g.reshape(1, -1), beta.reshape(1, -1))


def kernel(x, edge_index, edge_attr, batch, W_in, b_in, W_e, b_e, W_l, b_l,
           W1, b1, W2, b2, W3, b3, ln_g, ln_b):
    n, d = x.shape
    e = edge_index.shape[1]
    de = edge_attr.shape[1]
    h = W_in.shape[1]
    nlayers = W_l.shape[0]
    ngraphs = 64

    s_tot = -(-e // (_NS * _K))
    s_tot = -(-s_tot // (2 * _NBUF)) * (2 * _NBUF)
    cht = _NS * s_tot
    e_pad = cht * _K
    pad = e_pad - e
    n_acc = -(-n // (8 * _NS)) * (8 * _NS)
    if n_acc == n:
        n_acc += 8 * _NS

    src_p = jnp.concatenate(
        [edge_index[0], jnp.zeros((pad,), jnp.int32)]).reshape(cht, 1, _K)
    dst_p = jnp.concatenate(
        [edge_index[1], jnp.full((pad,), n, jnp.int32)]).reshape(cht, 1, _K)
    sd_p = jnp.concatenate([src_p, dst_p], axis=1)
    iota_p = jnp.arange(e_pad, dtype=jnp.int32).reshape(cht, 1, _K)
    sd_e = jnp.concatenate([iota_p, dst_p], axis=1)
    attr_p = jnp.concatenate(
        [edge_attr, jnp.zeros((pad, de), edge_attr.dtype)])

    h0 = _mm_bias_relu(x, W_in, b_in, bm=1000)
    erows = _edge_rows(attr_p, W_e, b_e, e, bm=2048)

    sc_e = _make_sc_pass(n, n_acc, s_tot // 2, s_tot - s_tot // 2, h)
    ep = sc_e(erows, sd_e)

    g0 = (6 * s_tot // 7) // _NBUF * _NBUF
    sc_h = _make_sc_pass(n, n_acc, g0, s_tot - g0, h)
    hc = h0
    for l in range(nlayers):
        pp = sc_h(hc, sd_p)
        hc = _layer_update(hc, pp, ep, W_l[l], b_l[l], bm=1000)

    return _readout(hc, batch.reshape(1, n), W1, b1, W2, b2, W3, b3,
                    ln_g, ln_b, ngraphs)


# speedup vs baseline: 1.4615x; 1.0722x over previous
import functools

import jax
import jax.numpy as jnp
from jax import lax
from jax.experimental import pallas as pl
from jax.experimental.pallas import tpu as pltpu
from jax.experimental.pallas import tpu_sc as plsc

_NC = 2
_NS = 16
_NW = _NC * _NS
_K = 64
_NBUF = 4


def _make_sc_pass(n_rows, n_acc, g0, g1, feat):
    mesh = plsc.VectorSubcoreMesh(core_axis_name="c", subcore_axis_name="s")
    zr = n_acc // _NS
    wr = (n_rows // _NS) // 8 * 8
    wr_last = n_rows - (_NS - 1) * wr

    def body(table, sd_hbm, out, acc, *rest):
        sd = rest[0:_NBUF]
        st = rest[_NBUF:2 * _NBUF]
        isem = rest[2 * _NBUF:3 * _NBUF]
        gsem = rest[3 * _NBUF:4 * _NBUF]
        cc = lax.axis_index("c")
        ss = lax.axis_index("s")

        st0 = st[0]

        def zrow(i, carry):
            for j in range(feat // 16):
                st0[i, pl.ds(j * 16, 16)] = jnp.zeros((16,), jnp.float32)
            return carry
        lax.fori_loop(0, _K, zrow, 0)
        nfull, rem = zr // _K, zr % _K
        for t in range(nfull):
            pltpu.sync_copy(st0, acc.at[pl.ds(ss * zr + t * _K, _K), :])
        if rem:
            pltpu.sync_copy(st0.at[pl.ds(0, rem), :],
                            acc.at[pl.ds(ss * zr + nfull * _K, rem), :])
        plsc.subcore_barrier()

        def issue_idx(c, slot):
            pltpu.async_copy(sd_hbm.at[c], sd[slot], isem[slot])

        def wait_idx(slot):
            pltpu.make_async_copy(sd_hbm.at[0], sd[slot],
                                  isem[slot]).wait()

        def issue_gather(slot):
            pltpu.async_copy(table.at[sd[slot].at[0]], st[slot], gsem[slot])

        def wait_gather(slot):
            pltpu.make_async_copy(table.at[sd[slot].at[0]], st[slot],
                                  gsem[slot]).wait()

        def run(base, g):
            for b in range(_NBUF):
                issue_idx(base + b, b)
            for b in range(2):
                wait_idx(b)
                issue_gather(b)

            def group(i, carry):
                for b in range(_NBUF):
                    c = _NBUF * i + b
                    c2 = c + 2

                    @pl.when(c2 < g)
                    def _():
                        wait_idx((b + 2) % _NBUF)
                        issue_gather((b + 2) % _NBUF)
                    wait_gather(b)
                    pltpu.sync_copy(st[b], acc.at[sd[b].at[1]], add=True)

                    @pl.when(c + _NBUF < g)
                    def _():
                        issue_idx(base + c + _NBUF, b)
                return carry
            lax.fori_loop(0, g // _NBUF, group, 0)

        @pl.when(cc == 0)
        def _():
            run(ss * g0, g0)

        @pl.when(cc == 1)
        def _():
            run(_NS * g0 + ss * g1, g1)
        plsc.subcore_barrier()

        @pl.when(ss < _NS - 1)
        def _():
            pltpu.sync_copy(acc.at[pl.ds(ss * wr, wr), :],
                            out.at[cc, pl.ds(ss * wr, wr), :])

        @pl.when(ss == _NS - 1)
        def _():
            pltpu.sync_copy(acc.at[pl.ds((_NS - 1) * wr, wr_last), :],
                            out.at[cc, pl.ds((_NS - 1) * wr, wr_last), :])

    scratch = (
        [pltpu.VMEM_SHARED((n_acc, feat), jnp.float32)]
        + [pltpu.VMEM((2, _K), jnp.int32)] * _NBUF
        + [pltpu.VMEM((_K, feat), jnp.float32)] * _NBUF
        + [pltpu.SemaphoreType.DMA] * (2 * _NBUF)
    )
    return pl.kernel(
        body,
        out_type=jax.ShapeDtypeStruct((_NC, n_rows, feat), jnp.float32),
        mesh=mesh,
        scratch_types=scratch,
    )


def _mm_bias_relu(x, w, b, bm):
    m, d = x.shape
    h = w.shape[1]

    def body(x_ref, w_ref, b_ref, o_ref):
        z = jnp.dot(x_ref[:, :], w_ref[:, :],
                    preferred_element_type=jnp.float32) + b_ref[0:1, :]
        o_ref[:, :] = jnp.maximum(z, 0.0)

    return pl.pallas_call(
        body,
        grid=(m // bm,),
        in_specs=[
            pl.BlockSpec((bm, d), lambda i: (i, 0)),
            pl.BlockSpec((d, h), lambda i: (0, 0)),
            pl.BlockSpec((1, h), lambda i: (0, 0)),
        ],
        out_specs=pl.BlockSpec((bm, h), lambda i: (i, 0)),
        out_shape=jax.ShapeDtypeStruct((m, h), jnp.float32),
    )(x, w, b.reshape(1, h))


def _edge_rows(attr, w, b, n_real, bm):
    m, de = attr.shape
    h = w.shape[1]

    def body(a_ref, w_ref, b_ref, o_ref):
        i = pl.program_id(0)
        z = jnp.dot(a_ref[:, :], w_ref[:, :],
                    preferred_element_type=jnp.float32) + b_ref[0:1, :]
        z = jnp.maximum(z, 0.0)
        rows = i * bm + lax.broadcasted_iota(jnp.int32, z.shape, 0)
        o_ref[:, :] = jnp.where(rows < n_real, z, 0.0)

    return pl.pallas_call(
        body,
        grid=(m // bm,),
        in_specs=[
            pl.BlockSpec((bm, de), lambda i: (i, 0)),
            pl.BlockSpec((de, h), lambda i: (0, 0)),
            pl.BlockSpec((1, h), lambda i: (0, 0)),
        ],
        out_specs=pl.BlockSpec((bm, h), lambda i: (i, 0)),
        out_shape=jax.ShapeDtypeStruct((m, h), jnp.float32),
    )(attr, w, b.reshape(1, h))


def _layer_update(hc, pp, ep, w, b, bm):
    m, h = hc.shape

    def body(h_ref, p0_ref, p1_ref, q0_ref, q1_ref, w_ref, b_ref, o_ref):
        s = (p0_ref[:, :] + p1_ref[:, :]) + (q0_ref[:, :] + q1_ref[:, :])
        z = jnp.dot(s, w_ref[:, :],
                    preferred_element_type=jnp.float32) + b_ref[0:1, :]
        o_ref[:, :] = h_ref[:, :] + jnp.maximum(z, 0.0)

    blk = pl.BlockSpec((bm, h), lambda i: (i, 0))
    return pl.pallas_call(
        body,
        grid=(m // bm,),
        in_specs=[blk, blk, blk, blk, blk,
                  pl.BlockSpec((h, h), lambda i: (0, 0)),
                  pl.BlockSpec((1, h), lambda i: (0, 0))],
        out_specs=blk,
        out_shape=jax.ShapeDtypeStruct((m, h), jnp.float32),
    )(hc, pp[0], pp[1], ep[0], ep[1], w, b.reshape(1, h))


def _readout(hc, batch2d, w1, b1, w2, b2, w3, b3, g, beta, ngraphs):
    m, h = hc.shape
    out = w3.shape[1]

    def body(h_ref, bt_ref, w1_ref, b1_ref, w2_ref, b2_ref, w3_ref, b3_ref,
             g_ref, be_ref, o_ref):
        gids = lax.broadcasted_iota(jnp.int32, (ngraphs, m), 0)
        onehot = (bt_ref[:, :] == gids).astype(jnp.float32)
        cnt = jnp.sum(onehot, axis=1, keepdims=True)
        sums = jnp.dot(onehot, h_ref[:, :], preferred_element_type=jnp.float32)
        hg = sums / jnp.maximum(cnt, 1.0)
        z = jnp.maximum(jnp.dot(hg, w1_ref[:, :],
                                preferred_element_type=jnp.float32)
                        + b1_ref[0:1, :], 0.0)
        z = jnp.maximum(jnp.dot(z, w2_ref[:, :],
                                preferred_element_type=jnp.float32)
                        + b2_ref[0:1, :], 0.0)
        z = jnp.dot(z, w3_ref[:, :],
                    preferred_element_type=jnp.float32) + b3_ref[0:1, :]
        mu = jnp.mean(z, axis=-1, keepdims=True)
        var = jnp.mean((z - mu) ** 2, axis=-1, keepdims=True)
        o_ref[:, :] = ((z - mu) / jnp.sqrt(var + 1e-5)) * g_ref[0:1, :] \
            + be_ref[0:1, :]

    return pl.pallas_call(
        body,
        out_shape=jax.ShapeDtypeStruct((ngraphs, out), jnp.float32),
    )(hc, batch2d, w1, b1.reshape(1, -1), w2, b2.reshape(1, -1),
      w3, b3.reshape(1, -1), g.reshape(1, -1), beta.reshape(1, -1))


def kernel(x, edge_index, edge_attr, batch, W_in, b_in, W_e, b_e, W_l, b_l,
           W1, b1, W2, b2, W3, b3, ln_g, ln_b):
    n, d = x.shape
    e = edge_index.shape[1]
    de = edge_attr.shape[1]
    h = W_in.shape[1]
    nlayers = W_l.shape[0]
    ngraphs = 64

    s_tot = -(-e // (_NS * _K))
    s_tot = -(-s_tot // (2 * _NBUF)) * (2 * _NBUF)
    cht = _NS * s_tot
    e_pad = cht * _K
    pad = e_pad - e
    n_acc = -(-n // (8 * _NS)) * (8 * _NS)
    if n_acc == n:
        n_acc += 8 * _NS

    src_p = jnp.concatenate(
        [edge_index[0], jnp.zeros((pad,), jnp.int32)]).reshape(cht, 1, _K)
    dst_p = jnp.concatenate(
        [edge_index[1], jnp.full((pad,), n, jnp.int32)]).reshape(cht, 1, _K)
    sd_p = jnp.concatenate([src_p, dst_p], axis=1)
    iota_p = jnp.arange(e_pad, dtype=jnp.int32).reshape(cht, 1, _K)
    sd_e = jnp.concatenate([iota_p, dst_p], axis=1)
    attr_p = jnp.concatenate(
        [edge_attr, jnp.zeros((pad, de), edge_attr.dtype)])

    h0 = _mm_bias_relu(x, W_in, b_in, bm=1000)
    erows = _edge_rows(attr_p, W_e, b_e, e, bm=2048)

    sc_e = _make_sc_pass(n, n_acc, s_tot // 2, s_tot - s_tot // 2, h)
    ep = sc_e(erows, sd_e)

    g0 = (s_tot - 8) // _NBUF * _NBUF
    sc_h = _make_sc_pass(n, n_acc, g0, s_tot - g0, h)
    hc = h0
    for l in range(nlayers):
        pp = sc_h(hc, sd_p)
        hc = _layer_update(hc, pp, ep, W_l[l], b_l[l], bm=1000)

    return _readout(hc, batch.reshape(1, n), W1, b1, W2, b2, W3, b3,
                    ln_g, ln_b, ngraphs)
